# baseline (speedup 1.0000x reference)
import jax
import jax.numpy as jnp
from jax import lax
from jax.experimental import pallas as pl
from jax.experimental.pallas import tpu as pltpu
from jax.experimental.pallas import tpu_sc as plsc

BATCH = 8192
DIM = 2048
NC = 2
NS = 16
NW = NC * NS
LANES = 16
ROWS_PER_W = BATCH // NW
R = 8
CHUNKS = ROWS_PER_W // R
JGROUPS = DIM // LANES


def _body(in_hbm, perm_hbm, out_hbm,
          perm_v, in0, in1, out0, out1, si0, si1, so0, so1):
    wid = lax.axis_index("s") * NC + lax.axis_index("c")
    row0 = wid * ROWS_PER_W

    pltpu.sync_copy(perm_hbm, perm_v)

    def in_cp(c, buf, sem):
        return pltpu.make_async_copy(
            in_hbm.at[pl.ds(row0 + c * R, R), :], buf, sem)

    def out_cp(c, buf, sem):
        return pltpu.make_async_copy(
            buf, out_hbm.at[pl.ds(row0 + c * R, R), :], sem)

    in_cp(0, in0, si0).start()
    in_cp(1, in1, si1).start()

    def gather_chunk(inbuf, outbuf):
        @plsc.parallel_loop(0, JGROUPS, unroll=8)
        def jg_body(jg):
            idx = perm_v[pl.ds(jg * LANES, LANES)]
            for r in range(R):
                row_idx = jnp.full((LANES,), r, jnp.int32)
                g = plsc.load_gather(inbuf, [row_idx, idx])
                outbuf[r, pl.ds(jg * LANES, LANES)] = g

    def super_body(k, carry):
        for b, (ib, ob, si, so) in enumerate(
                ((in0, out0, si0, so0), (in1, out1, si1, so1))):
            c = 2 * k + b
            in_cp(c, ib, si).wait()

            @pl.when(k > 0)
            def _wait_out():
                out_cp(c - 2, ob, so).wait()

            gather_chunk(ib, ob)
            out_cp(c, ob, so).start()

            @pl.when(k < (CHUNKS // 2 - 1))
            def _start_next_in():
                in_cp(c + 2, ib, si).start()
        return carry

    lax.fori_loop(0, CHUNKS // 2, super_body, None)
    out_cp(CHUNKS - 2, out0, so0).wait()
    out_cp(CHUNKS - 1, out1, so1).wait()


@jax.jit
def kernel(inputs, perm):
    permute = pl.kernel(
        _body,
        out_type=jax.ShapeDtypeStruct((BATCH, DIM), jnp.float32),
        mesh=plsc.VectorSubcoreMesh(core_axis_name="c", subcore_axis_name="s"),
        compiler_params=pltpu.CompilerParams(needs_layout_passes=False),
        scratch_types=[
            pltpu.VMEM((DIM,), jnp.int32),
            pltpu.VMEM((R, DIM), jnp.float32),
            pltpu.VMEM((R, DIM), jnp.float32),
            pltpu.VMEM((R, DIM), jnp.float32),
            pltpu.VMEM((R, DIM), jnp.float32),
            pltpu.SemaphoreType.DMA,
            pltpu.SemaphoreType.DMA,
            pltpu.SemaphoreType.DMA,
            pltpu.SemaphoreType.DMA,
        ],
    )
    out = permute(inputs, perm.astype(jnp.int32))
    logdet = jnp.zeros((BATCH,), jnp.float32)
    return (out, logdet)


# speedup vs baseline: 1.0308x; 1.0308x over previous
import jax
import jax.numpy as jnp
from jax import lax
from jax.experimental import pallas as pl
from jax.experimental.pallas import tpu as pltpu
from jax.experimental.pallas import tpu_sc as plsc

BATCH = 8192
DIM = 2048
NC = 2
NS = 16
NW = NC * NS
LANES = 16
ROWS_PER_W = BATCH // NW
R = 4
CHUNKS = ROWS_PER_W // R
NBUF = 4
JGROUPS = DIM // LANES


def _body(in_hbm, perm_hbm, out_hbm, perm_v, *bufs):
    in_v = bufs[0:NBUF]
    out_v = bufs[NBUF:2 * NBUF]
    si = bufs[2 * NBUF:3 * NBUF]
    so = bufs[3 * NBUF:4 * NBUF]

    wid = lax.axis_index("s") * NC + lax.axis_index("c")
    row0 = wid * ROWS_PER_W

    def in_cp(c, b):
        return pltpu.make_async_copy(
            in_hbm.at[pl.ds(row0 + c * R, R), :], in_v[b], si[b])

    def out_cp(c, b):
        return pltpu.make_async_copy(
            out_v[b], out_hbm.at[pl.ds(row0 + c * R, R), :], so[b])

    for b in range(NBUF):
        in_cp(b, b).start()

    pltpu.sync_copy(perm_hbm, perm_v)

    def gather_chunk(inbuf, outbuf):
        @plsc.parallel_loop(0, JGROUPS, unroll=4)
        def jg_body(jg):
            idx = perm_v[pl.ds(jg * LANES, LANES)]
            for r in range(R):
                row_idx = jnp.full((LANES,), r, jnp.int32)
                g = plsc.load_gather(inbuf, [row_idx, idx])
                outbuf[r, pl.ds(jg * LANES, LANES)] = g

    def super_body(k, carry):
        for b in range(NBUF):
            c = k * NBUF + b
            in_cp(c, b).wait()

            @pl.when(k > 0)
            def _wait_out():
                out_cp(c - NBUF, b).wait()

            gather_chunk(in_v[b], out_v[b])
            out_cp(c, b).start()

            @pl.when(k < (CHUNKS // NBUF - 1))
            def _start_next_in():
                in_cp(c + NBUF, b).start()
        return carry

    lax.fori_loop(0, CHUNKS // NBUF, super_body, None)
    for b in range(NBUF):
        out_cp(CHUNKS - NBUF + b, b).wait()


@jax.jit
def kernel(inputs, perm):
    permute = pl.kernel(
        _body,
        out_type=jax.ShapeDtypeStruct((BATCH, DIM), jnp.float32),
        mesh=plsc.VectorSubcoreMesh(core_axis_name="c", subcore_axis_name="s"),
        compiler_params=pltpu.CompilerParams(needs_layout_passes=False),
        scratch_types=(
            [pltpu.VMEM((DIM,), jnp.int32)]
            + [pltpu.VMEM((R, DIM), jnp.float32) for _ in range(2 * NBUF)]
            + [pltpu.SemaphoreType.DMA for _ in range(2 * NBUF)]
        ),
    )
    out = permute(inputs, perm.astype(jnp.int32))
    logdet = jnp.zeros((BATCH,), jnp.float32)
    return (out, logdet)


# speedup vs baseline: 1.0348x; 1.0039x over previous
import jax
import jax.numpy as jnp
from jax import lax
from jax.experimental import pallas as pl
from jax.experimental.pallas import tpu as pltpu
from jax.experimental.pallas import tpu_sc as plsc

BATCH = 8192
DIM = 2048
NC = 2
NS = 16
NW = NC * NS
LANES = 16
ROWS_PER_W = BATCH // NW
R = 2
CHUNKS = ROWS_PER_W // R
NBUF = 8
JGROUPS = DIM // LANES


def _body(in_hbm, perm_hbm, out_hbm, perm_v, *bufs):
    in_v = bufs[0:NBUF]
    out_v = bufs[NBUF:2 * NBUF]
    si = bufs[2 * NBUF:3 * NBUF]
    so = bufs[3 * NBUF:4 * NBUF]

    wid = lax.axis_index("s") * NC + lax.axis_index("c")
    row0 = wid * ROWS_PER_W

    def in_cp(c, b):
        return pltpu.make_async_copy(
            in_hbm.at[pl.ds(row0 + c * R, R), :], in_v[b], si[b])

    def out_cp(c, b):
        return pltpu.make_async_copy(
            out_v[b], out_hbm.at[pl.ds(row0 + c * R, R), :], so[b])

    for b in range(NBUF):
        in_cp(b, b).start()

    pltpu.sync_copy(perm_hbm, perm_v)

    def gather_chunk(inbuf, outbuf):
        @plsc.parallel_loop(0, JGROUPS, unroll=4)
        def jg_body(jg):
            idx = perm_v[pl.ds(jg * LANES, LANES)]
            for r in range(R):
                row_idx = jnp.full((LANES,), r, jnp.int32)
                g = plsc.load_gather(inbuf, [row_idx, idx])
                outbuf[r, pl.ds(jg * LANES, LANES)] = g

    def super_body(k, carry):
        for b in range(NBUF):
            c = k * NBUF + b
            in_cp(c, b).wait()

            @pl.when(k > 0)
            def _wait_out():
                out_cp(c - NBUF, b).wait()

            gather_chunk(in_v[b], out_v[b])
            out_cp(c, b).start()

            @pl.when(k < (CHUNKS // NBUF - 1))
            def _start_next_in():
                in_cp(c + NBUF, b).start()
        return carry

    lax.fori_loop(0, CHUNKS // NBUF, super_body, None)
    for b in range(NBUF):
        out_cp(CHUNKS - NBUF + b, b).wait()


@jax.jit
def kernel(inputs, perm):
    permute = pl.kernel(
        _body,
        out_type=jax.ShapeDtypeStruct((BATCH, DIM), jnp.float32),
        mesh=plsc.VectorSubcoreMesh(core_axis_name="c", subcore_axis_name="s"),
        compiler_params=pltpu.CompilerParams(needs_layout_passes=False),
        scratch_types=(
            [pltpu.VMEM((DIM,), jnp.int32)]
            + [pltpu.VMEM((R, DIM), jnp.float32) for _ in range(2 * NBUF)]
            + [pltpu.SemaphoreType.DMA for _ in range(2 * NBUF)]
        ),
    )
    out = permute(inputs, perm.astype(jnp.int32))
    logdet = jnp.zeros((BATCH,), jnp.float32)
    return (out, logdet)
